# baseline (speedup 1.0000x reference)
import functools

import jax
import jax.numpy as jnp
from jax import lax
from jax.experimental import pallas as pl
from jax.experimental.pallas import tpu as pltpu
from jax.experimental.pallas import tpu_sc as plsc

N_GRAPHS = 512
SUB_ROWS = 512
ROWS_PER_BLOCK = 8 * SUB_ROWS
N_WORK = 25
ROWS_W = 8
CHUNKS_W = ROWS_W * SUB_ROWS // 128
BIN_PAD = 528


def _tc_dot_body(x_ref, w_ref, o_ref):
    w = w_ref[...]
    for j in range(8):
        s = jax.lax.dot_general(w, x_ref[pl.ds(j * SUB_ROWS, SUB_ROWS), :],
                                (((1,), (1,)), ((), ())),
                                preferred_element_type=jnp.float32)
        o_ref[0, j, :] = s[0]


def _tc_combine_body(p_ref, o_ref):
    o_ref[...] = p_ref[0:1, :] + p_ref[1:2, :]


def _sc_segment_sum(y2, b1, n_nodes):
    mesh = plsc.VectorSubcoreMesh(core_axis_name="c", subcore_axis_name="s",
                                  num_cores=2, num_subcores=16)
    per_w = ROWS_W * SUB_ROWS
    tail_n = n_nodes - (N_WORK - 1) * per_w
    tail_full = tail_n // 128
    tail_rem = tail_n - tail_full * 128

    @functools.partial(
        pl.kernel,
        out_type=jax.ShapeDtypeStruct((2 * N_GRAPHS,), jnp.float32),
        mesh=mesh,
        scratch_types=[
            pltpu.VMEM((ROWS_W, SUB_ROWS), jnp.float32),
            pltpu.VMEM((CHUNKS_W, 128), jnp.int32),
            pltpu.VMEM((BIN_PAD,), jnp.float32),
            pltpu.VMEM_SHARED((BIN_PAD,), jnp.float32),
            pltpu.SemaphoreType.DMA,
            pltpu.SemaphoreType.DMA,
        ],
        compiler_params=pltpu.CompilerParams(needs_layout_passes=False),
    )
    def seg_sum(y_hbm, b_hbm, out_hbm, val_v, idx_v, zbuf_v, bins_sh,
                sem_in, sem_sc):
        c = lax.axis_index("c")
        s = lax.axis_index("s")
        wid = s * 2 + c

        def load(n_full, rem):
            cps = [pltpu.async_copy(y_hbm.at[pl.ds(wid * ROWS_W, ROWS_W)],
                                    val_v, sem_in)]
            base = wid * per_w
            for j in range(n_full):
                cps.append(pltpu.async_copy(
                    b_hbm.at[pl.ds(base + 128 * j, 128)],
                    idx_v.at[j], sem_in))
            if rem:
                cps.append(pltpu.async_copy(
                    b_hbm.at[pl.ds(base + 128 * n_full, rem)],
                    idx_v.at[n_full, pl.ds(0, rem)], sem_in))
            for cp in cps:
                cp.wait()

        def accumulate(n_chunks):
            cps = [pltpu.async_copy(
                       val_v.at[j // 4, pl.ds(128 * (j % 4), 128)],
                       bins_sh.at[idx_v.at[j]], sem_sc, add=True)
                   for j in range(n_chunks)]
            for cp in cps:
                cp.wait()

        @pl.when(s == 0)
        def _zero_shared():
            for k in range(BIN_PAD // 16):
                zbuf_v[pl.ds(16 * k, 16)] = jnp.zeros((16,), jnp.float32)
            pltpu.sync_copy(zbuf_v, bins_sh)

        @pl.when(wid < N_WORK - 1)
        def _load_full():
            load(CHUNKS_W, 0)

        @pl.when(wid == N_WORK - 1)
        def _load_tail():
            for j in range(tail_rem // 16, 8):
                idx_v[tail_full, pl.ds(16 * j, 16)] = jnp.full(
                    (16,), N_GRAPHS, jnp.int32)
            load(tail_full, tail_rem)

        plsc.subcore_barrier()

        @pl.when(wid < N_WORK - 1)
        def _acc_full():
            accumulate(CHUNKS_W)

        @pl.when(wid == N_WORK - 1)
        def _acc_tail():
            accumulate(tail_full + 1)

        plsc.subcore_barrier()

        @pl.when(s == 0)
        def _write_out():
            pltpu.sync_copy(bins_sh.at[pl.ds(0, N_GRAPHS)],
                            out_hbm.at[pl.ds(N_GRAPHS * c, N_GRAPHS)])

    return seg_sum(y2, b1)


def kernel(node_features, batch, W):
    n, d = node_features.shape
    n_blocks = -(-n // ROWS_PER_BLOCK)
    n_pad = n_blocks * ROWS_PER_BLOCK

    y3 = pl.pallas_call(
        _tc_dot_body,
        grid=(n_blocks,),
        in_specs=[
            pl.BlockSpec((ROWS_PER_BLOCK, d), lambda i: (i, 0)),
            pl.BlockSpec((1, d), lambda i: (0, 0)),
        ],
        out_specs=pl.BlockSpec((1, 8, SUB_ROWS), lambda i: (i, 0, 0)),
        out_shape=jax.ShapeDtypeStruct((n_blocks, 8, SUB_ROWS), jnp.float32),
    )(node_features, W.reshape(1, d))
    y2 = y3.reshape(n_blocks * 8, SUB_ROWS)

    parts = _sc_segment_sum(y2, batch.astype(jnp.int32), n)

    out = pl.pallas_call(
        _tc_combine_body,
        in_specs=[pl.BlockSpec((2, N_GRAPHS), lambda: (0, 0))],
        out_specs=pl.BlockSpec((1, N_GRAPHS), lambda: (0, 0)),
        out_shape=jax.ShapeDtypeStruct((1, N_GRAPHS), jnp.float32),
    )(parts.reshape(2, N_GRAPHS))
    return out.reshape(N_GRAPHS, 1)


# speedup vs baseline: 1.1447x; 1.1447x over previous
import functools

import jax
import jax.numpy as jnp
from jax import lax
from jax.experimental import pallas as pl
from jax.experimental.pallas import tpu as pltpu
from jax.experimental.pallas import tpu_sc as plsc

N_GRAPHS = 512
SUB_ROWS = 896
ROWS_PER_BLOCK = 8 * SUB_ROWS
BIN_PAD = 528


def _tc_dot_body(x_ref, w_ref, o_ref):
    w = w_ref[...]
    for j in range(8):
        s = jax.lax.dot_general(w, x_ref[pl.ds(j * SUB_ROWS, SUB_ROWS), :],
                                (((1,), (1,)), ((), ())),
                                preferred_element_type=jnp.float32)
        for k in range(7):
            q = 7 * j + k
            o_ref[q // 8, q % 8, :] = s[0, 128 * k:128 * (k + 1)]


def _tc_combine_body(p_ref, o_ref):
    p = p_ref[...]
    o_ref[...] = p[:N_GRAPHS] + p[N_GRAPHS:]


def _sc_segment_sum(y2, b1, n_nodes):
    mesh = plsc.VectorSubcoreMesh(core_axis_name="c", subcore_axis_name="s",
                                  num_cores=2, num_subcores=16)
    rows_a, rows_b = 24, 32
    start_30 = 30 * rows_a
    start_31 = start_30 + rows_b
    tail_n = n_nodes - start_31 * 128
    tail_full = tail_n // 128
    tail_rem = tail_n - tail_full * 128

    @functools.partial(
        pl.kernel,
        out_type=jax.ShapeDtypeStruct((2 * N_GRAPHS,), jnp.float32),
        mesh=mesh,
        scratch_types=[
            pltpu.VMEM((rows_b, 128), jnp.float32),
            pltpu.VMEM((rows_b, 128), jnp.int32),
            pltpu.VMEM((BIN_PAD,), jnp.float32),
            pltpu.VMEM_SHARED((BIN_PAD,), jnp.float32),
            pltpu.SemaphoreType.DMA,
            pltpu.SemaphoreType.DMA,
        ],
        compiler_params=pltpu.CompilerParams(needs_layout_passes=False),
    )
    def seg_sum(y_hbm, b_hbm, out_hbm, val_v, idx_v, zbuf_v, bins_sh,
                sem_in, sem_sc):
        c = lax.axis_index("c")
        s = lax.axis_index("s")
        wid = s * 2 + c

        def load(row_start, n_rows, n_full, rem):
            cps = [pltpu.async_copy(y_hbm.at[pl.ds(row_start, n_rows)],
                                    val_v.at[pl.ds(0, n_rows)], sem_in)]
            base = row_start * 128
            for j in range(n_full):
                cps.append(pltpu.async_copy(
                    b_hbm.at[pl.ds(base + 128 * j, 128)],
                    idx_v.at[j], sem_in))
            if rem:
                cps.append(pltpu.async_copy(
                    b_hbm.at[pl.ds(base + 128 * n_full, rem)],
                    idx_v.at[n_full, pl.ds(0, rem)], sem_in))
            for cp in cps:
                cp.wait()

        def accumulate(n_chunks):
            cps = [pltpu.async_copy(val_v.at[j], bins_sh.at[idx_v.at[j]],
                                    sem_sc, add=True)
                   for j in range(n_chunks)]
            for cp in cps:
                cp.wait()

        @pl.when(s == 0)
        def _zero_shared():
            for k in range(BIN_PAD // 16):
                zbuf_v[pl.ds(16 * k, 16)] = jnp.zeros((16,), jnp.float32)
            pltpu.sync_copy(zbuf_v, bins_sh)

        @pl.when(wid < 30)
        def _load_a():
            load(wid * rows_a, rows_a, rows_a, 0)

        @pl.when(wid == 30)
        def _load_b():
            load(start_30, rows_b, rows_b, 0)

        @pl.when(wid == 31)
        def _load_tail():
            for j in range(tail_rem // 16, 8):
                idx_v[tail_full, pl.ds(16 * j, 16)] = jnp.full(
                    (16,), N_GRAPHS, jnp.int32)
            load(start_31, rows_b, tail_full, tail_rem)

        plsc.subcore_barrier()

        @pl.when(wid < 30)
        def _acc_a():
            accumulate(rows_a)

        @pl.when(wid == 30)
        def _acc_b():
            accumulate(rows_b)

        @pl.when(wid == 31)
        def _acc_tail():
            accumulate(tail_full + 1)

        plsc.subcore_barrier()

        @pl.when(s == 0)
        def _write_out():
            pltpu.sync_copy(bins_sh.at[pl.ds(0, N_GRAPHS)],
                            out_hbm.at[pl.ds(N_GRAPHS * c, N_GRAPHS)])

    return seg_sum(y2, b1)


def kernel(node_features, batch, W):
    n, d = node_features.shape
    n_blocks = -(-n // ROWS_PER_BLOCK)
    n_slabs = n_blocks * 7

    y3 = pl.pallas_call(
        _tc_dot_body,
        grid=(n_blocks,),
        in_specs=[
            pl.BlockSpec((ROWS_PER_BLOCK, d), lambda i: (i, 0)),
            pl.BlockSpec((1, d), lambda i: (0, 0)),
        ],
        out_specs=pl.BlockSpec((7, 8, 128), lambda i: (i, 0, 0)),
        out_shape=jax.ShapeDtypeStruct((n_slabs, 8, 128), jnp.float32),
    )(node_features, W.reshape(1, d))
    y2 = y3.reshape(n_slabs * 8, 128)

    parts = _sc_segment_sum(y2, batch.astype(jnp.int32), n)

    out = pl.pallas_call(
        _tc_combine_body,
        in_specs=[pl.BlockSpec((2 * N_GRAPHS,), lambda: (0,))],
        out_specs=pl.BlockSpec((N_GRAPHS,), lambda: (0,)),
        out_shape=jax.ShapeDtypeStruct((N_GRAPHS,), jnp.float32),
    )(parts)
    return out.reshape(N_GRAPHS, 1)
